# speedup vs baseline: 1.8941x; 1.8941x over previous
import functools

import jax
import jax.numpy as jnp
from jax.experimental import pallas as pl
from jax.experimental.pallas import tpu as pltpu

E = 8
T = 16384
D = 768
BM = 512


def _dense_body(sel_ref, rw_ref, x_ref, wt_ref, b_ref, o_ref):
    x = x_ref[...]
    sel = sel_ref[...]
    rw = rw_ref[...]
    acc = jnp.zeros((BM, D), dtype=jnp.float32)
    for e in range(E):
        w_e = jnp.where(sel == e, rw, 0.0)
        out_e = jnp.dot(x, wt_ref[e], preferred_element_type=jnp.float32)
        out_e = out_e + jnp.reshape(b_ref[e], (1, D))
        acc = acc + out_e * w_e
    o_ref[...] = acc


@jax.jit
def _dense(hidden_states, selected_experts, routing_weights, WT, b):
    grid = (T // BM,)
    return pl.pallas_call(
        _dense_body,
        grid=grid,
        in_specs=[
            pl.BlockSpec((BM, 1), lambda i: (i, 0)),
            pl.BlockSpec((BM, 1), lambda i: (i, 0)),
            pl.BlockSpec((BM, D), lambda i: (i, 0)),
            pl.BlockSpec((E, D, D), lambda i: (0, 0, 0)),
            pl.BlockSpec((E, D), lambda i: (0, 0)),
        ],
        out_specs=pl.BlockSpec((BM, D), lambda i: (i, 0)),
        out_shape=jax.ShapeDtypeStruct((T, D), jnp.float32),
    )(selected_experts, routing_weights, hidden_states, WT, b)


def kernel(hidden_states, selected_experts, routing_weights, W, b):
    WT = jnp.transpose(W, (0, 2, 1))
    return _dense(hidden_states, selected_experts, routing_weights, WT, b)


# speedup vs baseline: 2.0888x; 1.1028x over previous
import functools

import jax
import jax.numpy as jnp
from jax.experimental import pallas as pl
from jax.experimental.pallas import tpu as pltpu

E = 8
T = 16384
D = 768
BM = 512


def _dense_body(sel_ref, rw_ref, x_ref, wt_ref, b_ref, o_ref):
    x = x_ref[...].astype(jnp.bfloat16)
    sel = sel_ref[...]
    rw = rw_ref[...]
    acc = jnp.zeros((BM, D), dtype=jnp.float32)
    for e in range(E):
        w_e = jnp.where(sel == e, rw, 0.0)
        out_e = jnp.dot(x, wt_ref[e], preferred_element_type=jnp.float32)
        out_e = out_e + jnp.reshape(b_ref[e], (1, D))
        acc = acc + out_e * w_e
    o_ref[...] = acc


@jax.jit
def _dense(hidden_states, selected_experts, routing_weights, WT, b):
    grid = (T // BM,)
    return pl.pallas_call(
        _dense_body,
        grid=grid,
        in_specs=[
            pl.BlockSpec((BM, 1), lambda i: (i, 0)),
            pl.BlockSpec((BM, 1), lambda i: (i, 0)),
            pl.BlockSpec((BM, D), lambda i: (i, 0)),
            pl.BlockSpec((E, D, D), lambda i: (0, 0, 0)),
            pl.BlockSpec((E, D), lambda i: (0, 0)),
        ],
        out_specs=pl.BlockSpec((BM, D), lambda i: (i, 0)),
        out_shape=jax.ShapeDtypeStruct((T, D), jnp.float32),
    )(selected_experts, routing_weights, hidden_states, WT, b)


def kernel(hidden_states, selected_experts, routing_weights, W, b):
    WT = jnp.transpose(W, (0, 2, 1)).astype(jnp.bfloat16)
    return _dense(hidden_states, selected_experts, routing_weights, WT, b)
